# baseline (speedup 1.0000x reference)
import jax
import jax.numpy as jnp
from jax.experimental import pallas as pl
from jax.experimental.pallas import tpu as pltpu

_N_SLICES = 5


def _linear_body(x1_ref, x2_ref, wt_ref, b_ref, y_ref):
    h = wt_ref.shape[1]
    y1 = jnp.dot(x1_ref[:], wt_ref[:], preferred_element_type=jnp.float32)
    y2 = jnp.dot(x2_ref[:], wt_ref[:], preferred_element_type=jnp.float32)
    y_ref[:, :h] = (y1 + b_ref[:]).astype(jnp.bfloat16)
    y_ref[:, h:] = (y2 + b_ref[:]).astype(jnp.bfloat16)


def _agg_body(*refs):
    adj_refs = refs[:_N_SLICES]
    y_ref, prelu_ref, h_ref, csum_ref = refs[_N_SLICES:]
    hdim = csum_ref.shape[2]
    m = adj_refs[0].shape[0]
    p = prelu_ref[0, 0]
    cs = None
    for j, aref in enumerate(adj_refs):
        acc = jnp.dot(aref[:].astype(jnp.bfloat16), y_ref[:],
                      preferred_element_type=jnp.float32)
        hj = jnp.where(acc >= 0, acc, p * acc)
        h_ref[j * m:(j + 1) * m, :] = hj.astype(jnp.bfloat16)
        part = jnp.sum(hj[:, :hdim], axis=0)
        cs = part if cs is None else cs + part
    csum_ref[0, 0, :] = cs


def _score_body(h_ref, csum_ref, wb_ref, bias_ref, n_ref, s1_ref, s2_ref):
    hdim = wb_ref.shape[0]
    total = jnp.sum(csum_ref[:, 0, :], axis=0, keepdims=True)
    s = jax.nn.sigmoid(total * n_ref[0, 0])
    u = jnp.dot(s, wb_ref[:].T, preferred_element_type=jnp.float32)
    bias = bias_ref[0, 0]
    s1_ref[:] = jnp.sum(h_ref[:, :hdim] * u, axis=1, keepdims=True) + bias
    s2_ref[:] = jnp.sum(h_ref[:, hdim:] * u, axis=1, keepdims=True) + bias


def kernel(x_1, x_2, adj, W, b, prelu_w, Wb, bias_b):
    n = adj.shape[0]
    f_in = x_1.shape[2]
    h_dim = W.shape[0]

    x1 = x_1[0]
    x2 = x_2[0]
    wt = W.T
    b2 = b.reshape(1, h_dim)
    prelu2 = prelu_w.reshape(1, 1)
    wb2 = Wb[0]
    bias2 = bias_b.reshape(1, 1)
    inv_n = jnp.full((1, 1), 1.0 / n, dtype=jnp.float32)

    bm1 = 2000
    bm2 = 400
    g1 = n // bm1
    g2 = n // bm2

    y = pl.pallas_call(
        _linear_body,
        grid=(g1,),
        in_specs=[
            pl.BlockSpec((bm1, f_in), lambda i: (i, 0)),
            pl.BlockSpec((bm1, f_in), lambda i: (i, 0)),
            pl.BlockSpec((f_in, h_dim), lambda i: (0, 0)),
            pl.BlockSpec((1, h_dim), lambda i: (0, 0)),
        ],
        out_specs=pl.BlockSpec((bm1, 2 * h_dim), lambda i: (i, 0)),
        out_shape=jax.ShapeDtypeStruct((n, 2 * h_dim), jnp.bfloat16),
        compiler_params=pltpu.CompilerParams(
            dimension_semantics=(pltpu.PARALLEL,)),
    )(x1, x2, wt, b2)

    h, csum = pl.pallas_call(
        _agg_body,
        grid=(g2,),
        in_specs=[
            pl.BlockSpec((bm2 // _N_SLICES, n),
                         (lambda j: (lambda i: (_N_SLICES * i + j, 0)))(j))
            for j in range(_N_SLICES)
        ] + [
            pl.BlockSpec((n, 2 * h_dim), lambda i: (0, 0)),
            pl.BlockSpec((1, 1), lambda i: (0, 0)),
        ],
        out_specs=[
            pl.BlockSpec((bm2, 2 * h_dim), lambda i: (i, 0)),
            pl.BlockSpec((1, 1, h_dim), lambda i: (i, 0, 0)),
        ],
        out_shape=[
            jax.ShapeDtypeStruct((n, 2 * h_dim), jnp.bfloat16),
            jax.ShapeDtypeStruct((g2, 1, h_dim), jnp.float32),
        ],
        compiler_params=pltpu.CompilerParams(
            dimension_semantics=(pltpu.PARALLEL,)),
    )(*([adj] * _N_SLICES), y, prelu2)

    s1, s2 = pl.pallas_call(
        _score_body,
        grid=(g1,),
        in_specs=[
            pl.BlockSpec((bm1, 2 * h_dim), lambda i: (i, 0)),
            pl.BlockSpec((g2, 1, h_dim), lambda i: (0, 0, 0)),
            pl.BlockSpec((h_dim, h_dim), lambda i: (0, 0)),
            pl.BlockSpec((1, 1), lambda i: (0, 0)),
            pl.BlockSpec((1, 1), lambda i: (0, 0)),
        ],
        out_specs=[
            pl.BlockSpec((bm1, 1), lambda i: (i, 0)),
            pl.BlockSpec((bm1, 1), lambda i: (i, 0)),
        ],
        out_shape=[
            jax.ShapeDtypeStruct((n, 1), jnp.float32),
            jax.ShapeDtypeStruct((n, 1), jnp.float32),
        ],
        compiler_params=pltpu.CompilerParams(
            dimension_semantics=(pltpu.PARALLEL,)),
    )(h, csum, wb2, bias2, inv_n)

    return jnp.concatenate([s1.reshape(1, n), s2.reshape(1, n)], axis=1)


# speedup vs baseline: 1.0824x; 1.0824x over previous
import jax
import jax.numpy as jnp
from jax.experimental import pallas as pl
from jax.experimental.pallas import tpu as pltpu


def _fused_body(x1_ref, x2_ref, wt_ref, b_ref, prelu_ref, wb_ref, bias_ref,
                invn_ref, adj_ref, s1_ref, s2_ref,
                y_ref, h_ref, csum_ref):
    i = pl.program_id(0)
    g = pl.num_programs(0) - 1
    hdim = wt_ref.shape[1]
    m = adj_ref.shape[0]

    @pl.when(i == 0)
    def _linear():
        y1 = jnp.dot(x1_ref[:].astype(jnp.bfloat16),
                     wt_ref[:].astype(jnp.bfloat16),
                     preferred_element_type=jnp.float32)
        y2 = jnp.dot(x2_ref[:].astype(jnp.bfloat16),
                     wt_ref[:].astype(jnp.bfloat16),
                     preferred_element_type=jnp.float32)
        y_ref[:, :hdim] = (y1 + b_ref[:]).astype(jnp.bfloat16)
        y_ref[:, hdim:] = (y2 + b_ref[:]).astype(jnp.bfloat16)
        csum_ref[:] = jnp.zeros_like(csum_ref)

    @pl.when(i < g)
    def _aggregate():
        p = prelu_ref[0, 0]
        acc = jnp.dot(adj_ref[:].astype(jnp.bfloat16), y_ref[:],
                      preferred_element_type=jnp.float32)
        hj = jnp.where(acc >= 0, acc, p * acc)
        h_ref[pl.ds(i * m, m), :] = hj.astype(jnp.bfloat16)
        csum_ref[0, :] = csum_ref[0, :] + jnp.sum(hj[:, :hdim], axis=0)

    @pl.when(i == g)
    def _score():
        s = jax.nn.sigmoid(csum_ref[:] * invn_ref[0, 0])
        u = jnp.dot(s, wb_ref[:].T, preferred_element_type=jnp.float32)
        bias = bias_ref[0, 0]
        s1_ref[:] = jnp.sum(h_ref[:, :hdim] * u, axis=1, keepdims=True) + bias
        s2_ref[:] = jnp.sum(h_ref[:, hdim:] * u, axis=1, keepdims=True) + bias


def kernel(x_1, x_2, adj, W, b, prelu_w, Wb, bias_b):
    n = adj.shape[0]
    f_in = x_1.shape[2]
    h_dim = W.shape[0]

    x1 = x_1[0]
    x2 = x_2[0]
    wt = W.T
    b2 = b.reshape(1, h_dim)
    prelu2 = prelu_w.reshape(1, 1)
    wb2 = Wb[0]
    bias2 = bias_b.reshape(1, 1)
    inv_n = jnp.full((1, 1), 1.0 / n, dtype=jnp.float32)

    bm = 200
    g = n // bm
    last = g - 1

    s1, s2 = pl.pallas_call(
        _fused_body,
        grid=(g + 1,),
        in_specs=[
            pl.BlockSpec((n, f_in), lambda i: (0, 0)),
            pl.BlockSpec((n, f_in), lambda i: (0, 0)),
            pl.BlockSpec((f_in, h_dim), lambda i: (0, 0)),
            pl.BlockSpec((1, h_dim), lambda i: (0, 0)),
            pl.BlockSpec((1, 1), lambda i: (0, 0)),
            pl.BlockSpec((h_dim, h_dim), lambda i: (0, 0)),
            pl.BlockSpec((1, 1), lambda i: (0, 0)),
            pl.BlockSpec((1, 1), lambda i: (0, 0)),
            pl.BlockSpec((bm, n),
                         lambda i: (jnp.minimum(i, last), 0)),
        ],
        out_specs=[
            pl.BlockSpec((n, 1), lambda i: (0, 0)),
            pl.BlockSpec((n, 1), lambda i: (0, 0)),
        ],
        out_shape=[
            jax.ShapeDtypeStruct((n, 1), jnp.float32),
            jax.ShapeDtypeStruct((n, 1), jnp.float32),
        ],
        scratch_shapes=[
            pltpu.VMEM((n, 2 * h_dim), jnp.bfloat16),
            pltpu.VMEM((n, 2 * h_dim), jnp.bfloat16),
            pltpu.VMEM((1, h_dim), jnp.float32),
        ],
        compiler_params=pltpu.CompilerParams(
            dimension_semantics=(pltpu.ARBITRARY,)),
    )(x1, x2, wt, b2, prelu2, wb2, bias2, inv_n, adj)

    return jnp.concatenate([s1.reshape(1, n), s2.reshape(1, n)], axis=1)


# speedup vs baseline: 1.0957x; 1.0123x over previous
import jax
import jax.numpy as jnp
from jax.experimental import pallas as pl
from jax.experimental.pallas import tpu as pltpu


def _fused_body(x1_ref, x2_ref, wt_ref, b_ref, prelu_ref, wb_ref, bias_ref,
                invn_ref, adj_ref, s1_ref, s2_ref,
                y_ref, h_ref, csum_ref):
    i = pl.program_id(0)
    g = pl.num_programs(0) - 1
    hdim = wt_ref.shape[1]
    m = adj_ref.shape[0]

    @pl.when(i == 0)
    def _linear():
        y1 = jnp.dot(x1_ref[:].astype(jnp.bfloat16),
                     wt_ref[:].astype(jnp.bfloat16),
                     preferred_element_type=jnp.float32)
        y2 = jnp.dot(x2_ref[:].astype(jnp.bfloat16),
                     wt_ref[:].astype(jnp.bfloat16),
                     preferred_element_type=jnp.float32)
        y_ref[:, :hdim] = (y1 + b_ref[:]).astype(jnp.bfloat16)
        y_ref[:, hdim:] = (y2 + b_ref[:]).astype(jnp.bfloat16)
        csum_ref[:] = jnp.zeros_like(csum_ref)

    @pl.when(i < g)
    def _aggregate():
        p = prelu_ref[0, 0]
        acc = jnp.dot(adj_ref[:].astype(jnp.bfloat16), y_ref[:],
                      preferred_element_type=jnp.float32)
        hj = jnp.where(acc >= 0, acc, p * acc)
        h_ref[pl.ds(i * m, m), :] = hj.astype(jnp.bfloat16)
        csum_ref[0, :] = csum_ref[0, :] + jnp.sum(hj[:, :hdim], axis=0)

    @pl.when(i == g)
    def _score():
        s = jax.nn.sigmoid(csum_ref[:] * invn_ref[0, 0])
        u = jnp.dot(s, wb_ref[:].T, preferred_element_type=jnp.float32)
        bias = bias_ref[0, 0]
        s1_ref[:] = jnp.sum(h_ref[:, :hdim] * u, axis=1, keepdims=True) + bias
        s2_ref[:] = jnp.sum(h_ref[:, hdim:] * u, axis=1, keepdims=True) + bias


def kernel(x_1, x_2, adj, W, b, prelu_w, Wb, bias_b):
    n = adj.shape[0]
    f_in = x_1.shape[2]
    h_dim = W.shape[0]

    x1 = x_1[0]
    x2 = x_2[0]
    wt = W.T
    b2 = b.reshape(1, h_dim)
    prelu2 = prelu_w.reshape(1, 1)
    wb2 = Wb[0]
    bias2 = bias_b.reshape(1, 1)
    inv_n = jnp.full((1, 1), 1.0 / n, dtype=jnp.float32)

    bm = 400
    g = n // bm
    last = g - 1

    s1, s2 = pl.pallas_call(
        _fused_body,
        grid=(g + 1,),
        in_specs=[
            pl.BlockSpec((n, f_in), lambda i: (0, 0)),
            pl.BlockSpec((n, f_in), lambda i: (0, 0)),
            pl.BlockSpec((f_in, h_dim), lambda i: (0, 0)),
            pl.BlockSpec((1, h_dim), lambda i: (0, 0)),
            pl.BlockSpec((1, 1), lambda i: (0, 0)),
            pl.BlockSpec((h_dim, h_dim), lambda i: (0, 0)),
            pl.BlockSpec((1, 1), lambda i: (0, 0)),
            pl.BlockSpec((1, 1), lambda i: (0, 0)),
            pl.BlockSpec((bm, n),
                         lambda i: (jnp.minimum(i, last), 0)),
        ],
        out_specs=[
            pl.BlockSpec((n, 1), lambda i: (0, 0)),
            pl.BlockSpec((n, 1), lambda i: (0, 0)),
        ],
        out_shape=[
            jax.ShapeDtypeStruct((n, 1), jnp.float32),
            jax.ShapeDtypeStruct((n, 1), jnp.float32),
        ],
        scratch_shapes=[
            pltpu.VMEM((n, 2 * h_dim), jnp.bfloat16),
            pltpu.VMEM((n, 2 * h_dim), jnp.bfloat16),
            pltpu.VMEM((1, h_dim), jnp.float32),
        ],
        compiler_params=pltpu.CompilerParams(
            dimension_semantics=(pltpu.ARBITRARY,),
            vmem_limit_bytes=64 * 1024 * 1024),
    )(x1, x2, wt, b2, prelu2, wb2, bias2, inv_n, adj)

    return jnp.concatenate([s1.reshape(1, n), s2.reshape(1, n)], axis=1)


# speedup vs baseline: 1.1035x; 1.0071x over previous
import jax
import jax.numpy as jnp
from jax.experimental import pallas as pl
from jax.experimental.pallas import tpu as pltpu

_BM = 200
_SLOTS = 4


def _fused_body(x1_ref, x2_ref, wt_ref, b_ref, prelu_ref, wb_ref, bias_ref,
                invn_ref, adj_ref, s1_ref, s2_ref,
                y_ref, h_ref, csum_ref, abuf_ref, sems):
    hdim = wt_ref.shape[1]
    n = x1_ref.shape[0]
    nblk = n // _BM

    def _adj_copy(blk, slot):
        return pltpu.make_async_copy(
            adj_ref.at[pl.ds(blk * _BM, _BM), :], abuf_ref.at[slot],
            sems.at[slot])

    for s in range(_SLOTS):
        _adj_copy(s, s).start()

    y1 = jnp.dot(x1_ref[:].astype(jnp.bfloat16),
                 wt_ref[:].astype(jnp.bfloat16),
                 preferred_element_type=jnp.float32)
    y2 = jnp.dot(x2_ref[:].astype(jnp.bfloat16),
                 wt_ref[:].astype(jnp.bfloat16),
                 preferred_element_type=jnp.float32)
    y_ref[:, :hdim] = (y1 + b_ref[:]).astype(jnp.bfloat16)
    y_ref[:, hdim:] = (y2 + b_ref[:]).astype(jnp.bfloat16)
    csum_ref[:] = jnp.zeros_like(csum_ref)

    p = prelu_ref[0, 0]

    def _block(ib, carry):
        slot = jax.lax.rem(ib, _SLOTS)
        _adj_copy(ib, slot).wait()
        acc = jnp.dot(abuf_ref[slot].astype(jnp.bfloat16), y_ref[:],
                      preferred_element_type=jnp.float32)
        hj = jnp.where(acc >= 0, acc, p * acc)
        h_ref[pl.ds(ib * _BM, _BM), :] = hj.astype(jnp.bfloat16)
        csum_ref[0, :] = csum_ref[0, :] + jnp.sum(hj[:, :hdim], axis=0)

        @pl.when(ib + _SLOTS < nblk)
        def _():
            _adj_copy(ib + _SLOTS, slot).start()
        return carry

    jax.lax.fori_loop(0, nblk, _block, 0)

    s = jax.nn.sigmoid(csum_ref[:] * invn_ref[0, 0])
    u = jnp.dot(s, wb_ref[:].T, preferred_element_type=jnp.float32)
    bias = bias_ref[0, 0]
    s1_ref[:] = jnp.sum(h_ref[:, :hdim] * u, axis=1, keepdims=True) + bias
    s2_ref[:] = jnp.sum(h_ref[:, hdim:] * u, axis=1, keepdims=True) + bias


def kernel(x_1, x_2, adj, W, b, prelu_w, Wb, bias_b):
    n = adj.shape[0]
    f_in = x_1.shape[2]
    h_dim = W.shape[0]

    x1 = x_1[0]
    x2 = x_2[0]
    wt = W.T
    b2 = b.reshape(1, h_dim)
    prelu2 = prelu_w.reshape(1, 1)
    wb2 = Wb[0]
    bias2 = bias_b.reshape(1, 1)
    inv_n = jnp.full((1, 1), 1.0 / n, dtype=jnp.float32)

    s1, s2 = pl.pallas_call(
        _fused_body,
        grid=(1,),
        in_specs=[
            pl.BlockSpec((n, f_in), lambda i: (0, 0)),
            pl.BlockSpec((n, f_in), lambda i: (0, 0)),
            pl.BlockSpec((f_in, h_dim), lambda i: (0, 0)),
            pl.BlockSpec((1, h_dim), lambda i: (0, 0)),
            pl.BlockSpec((1, 1), lambda i: (0, 0)),
            pl.BlockSpec((h_dim, h_dim), lambda i: (0, 0)),
            pl.BlockSpec((1, 1), lambda i: (0, 0)),
            pl.BlockSpec((1, 1), lambda i: (0, 0)),
            pl.BlockSpec(memory_space=pltpu.HBM),
        ],
        out_specs=[
            pl.BlockSpec((n, 1), lambda i: (0, 0)),
            pl.BlockSpec((n, 1), lambda i: (0, 0)),
        ],
        out_shape=[
            jax.ShapeDtypeStruct((n, 1), jnp.float32),
            jax.ShapeDtypeStruct((n, 1), jnp.float32),
        ],
        scratch_shapes=[
            pltpu.VMEM((n, 2 * h_dim), jnp.bfloat16),
            pltpu.VMEM((n, 2 * h_dim), jnp.bfloat16),
            pltpu.VMEM((1, h_dim), jnp.float32),
            pltpu.VMEM((_SLOTS, _BM, n), jnp.float32),
            pltpu.SemaphoreType.DMA((_SLOTS,)),
        ],
        compiler_params=pltpu.CompilerParams(
            dimension_semantics=(pltpu.ARBITRARY,),
            vmem_limit_bytes=64 * 1024 * 1024),
    )(x1, x2, wt, b2, prelu2, wb2, bias2, inv_n, adj)

    return jnp.concatenate([s1.reshape(1, n), s2.reshape(1, n)], axis=1)


# speedup vs baseline: 1.2135x; 1.0997x over previous
import jax
import jax.numpy as jnp
from jax.experimental import pallas as pl
from jax.experimental.pallas import tpu as pltpu


def _fused_body(x1_ref, x2_ref, wt_ref, b_ref, prelu_ref, wb_ref, bias_ref,
                invn_ref, adj_ref, out_ref,
                y_ref, h_ref, csum_ref):
    i = pl.program_id(0)
    g = pl.num_programs(0) - 1
    hdim = wt_ref.shape[1]
    m = adj_ref.shape[0]

    @pl.when(i == 0)
    def _linear():
        y1 = jnp.dot(x1_ref[:].astype(jnp.bfloat16),
                     wt_ref[:].astype(jnp.bfloat16),
                     preferred_element_type=jnp.float32)
        y2 = jnp.dot(x2_ref[:].astype(jnp.bfloat16),
                     wt_ref[:].astype(jnp.bfloat16),
                     preferred_element_type=jnp.float32)
        y_ref[:, :hdim] = (y1 + b_ref[:]).astype(jnp.bfloat16)
        y_ref[:, hdim:] = (y2 + b_ref[:]).astype(jnp.bfloat16)
        csum_ref[:] = jnp.zeros_like(csum_ref)

    @pl.when(i < g)
    def _aggregate():
        p = prelu_ref[0, 0]
        acc = jnp.dot(adj_ref[:].astype(jnp.bfloat16), y_ref[:],
                      preferred_element_type=jnp.float32)
        hj = jnp.where(acc >= 0, acc, p * acc)
        h_ref[pl.ds(i * m, m), :] = hj.astype(jnp.bfloat16)
        csum_ref[0, :] = csum_ref[0, :] + jnp.sum(hj[:, :hdim], axis=0)

    @pl.when(i == g)
    def _score():
        n = h_ref.shape[0]
        s = jax.nn.sigmoid(csum_ref[:] * invn_ref[0, 0])
        u = jnp.dot(s, wb_ref[:].T, preferred_element_type=jnp.float32)
        bias = bias_ref[0, 0]
        ub = u.astype(jnp.bfloat16)
        r1 = jax.lax.dot_general(ub, h_ref[:, :hdim],
                                 (((1,), (1,)), ((), ())),
                                 preferred_element_type=jnp.float32)
        r2 = jax.lax.dot_general(ub, h_ref[:, hdim:],
                                 (((1,), (1,)), ((), ())),
                                 preferred_element_type=jnp.float32)
        out_ref[0, :n] = r1[0, :] + bias
        out_ref[0, n:] = r2[0, :] + bias


def kernel(x_1, x_2, adj, W, b, prelu_w, Wb, bias_b):
    n = adj.shape[0]
    f_in = x_1.shape[2]
    h_dim = W.shape[0]

    x1 = x_1[0]
    x2 = x_2[0]
    wt = W.T
    b2 = b.reshape(1, h_dim)
    prelu2 = prelu_w.reshape(1, 1)
    wb2 = Wb[0]
    bias2 = bias_b.reshape(1, 1)
    inv_n = jnp.full((1, 1), 1.0 / n, dtype=jnp.float32)

    bm = 400
    g = n // bm
    last = g - 1

    logits = pl.pallas_call(
        _fused_body,
        grid=(g + 1,),
        in_specs=[
            pl.BlockSpec((n, f_in), lambda i: (0, 0)),
            pl.BlockSpec((n, f_in), lambda i: (0, 0)),
            pl.BlockSpec((f_in, h_dim), lambda i: (0, 0)),
            pl.BlockSpec((1, h_dim), lambda i: (0, 0)),
            pl.BlockSpec((1, 1), lambda i: (0, 0)),
            pl.BlockSpec((h_dim, h_dim), lambda i: (0, 0)),
            pl.BlockSpec((1, 1), lambda i: (0, 0)),
            pl.BlockSpec((1, 1), lambda i: (0, 0)),
            pl.BlockSpec((bm, n),
                         lambda i: (jnp.minimum(i, last), 0)),
        ],
        out_specs=pl.BlockSpec((1, 2 * n), lambda i: (0, 0)),
        out_shape=jax.ShapeDtypeStruct((1, 2 * n), jnp.float32),
        scratch_shapes=[
            pltpu.VMEM((n, 2 * h_dim), jnp.bfloat16),
            pltpu.VMEM((n, 2 * h_dim), jnp.bfloat16),
            pltpu.VMEM((1, h_dim), jnp.float32),
        ],
        compiler_params=pltpu.CompilerParams(
            dimension_semantics=(pltpu.ARBITRARY,),
            vmem_limit_bytes=64 * 1024 * 1024),
    )(x1, x2, wt, b2, prelu2, wb2, bias2, inv_n, adj)

    return logits
